# Initial kernel scaffold; baseline (speedup 1.0000x reference)
import jax
import jax.numpy as jnp
from jax.experimental import pallas as pl


def kernel(keyQ0, keyQ1, keyQ2, valueQ0, valueQ1, valueQ2, keyM0, keyM1, keyM2, valueM0, valueM1, valueM2, params):
    raise NotImplementedError("write your pallas kernel here")


# speedup vs baseline: 13.2059x; 13.2059x over previous
import functools
import math

import jax
import jax.numpy as jnp
from jax import lax
from jax.experimental import pallas as pl
from jax.experimental.pallas import tpu as pltpu

_B = 8
_CK = 64
_CV = 16
_K = 20
_SCALES = [(44, 44), (22, 22), (11, 11)]

_SQRT2 = math.sqrt(2.0)


def _gelu(x):
    return x * 0.5 * (1.0 + lax.erf(x / _SQRT2))


def _scale_body(N, Np, CB, nblk,
                mkt_ref, qk_ref, vqT_ref, vmt_ref, P2_ref, PT1_ref, P3_ref,
                sc_ref, out_ref, aff_s, work_s, pw_s):
    j = pl.program_id(1)
    f32 = jnp.float32

    mkt = mkt_ref[0]
    lhs = jnp.concatenate([mkt * 0.25, (mkt * mkt) * (-0.125)], axis=1)
    qk = qk_ref[0]
    rhs = jnp.concatenate([qk, jnp.ones((_CK, CB), f32)], axis=0)
    aff = lax.dot_general(lhs, rhs, (((1,), (0,)), ((), ())),
                          preferred_element_type=f32)
    rio = lax.broadcasted_iota(jnp.int32, (Np, 1), 0)
    aff = jnp.where(rio < N, aff, -1e30)
    aff_s[...] = aff
    work_s[...] = aff

    M = jnp.max(aff, axis=0, keepdims=True)

    def tk_body(_, c):
        S, _t = c
        w = work_s[...]
        cur = jnp.max(w, axis=0, keepdims=True)
        S = S + jnp.exp(cur - M)
        work_s[...] = jnp.where(w >= cur, -1e30, w)
        return (S, cur)

    S, t = lax.fori_loop(0, _K, tk_body,
                         (jnp.zeros((1, CB), f32), M))

    @pl.when(j == 0)
    def _():
        vqT = vqT_ref[0]
        pooled = jnp.sum(vqT, axis=1, keepdims=True) * (1.0 / N)
        h = jnp.sum(pooled * P2_ref[:, 0:1], axis=0, keepdims=True) + sc_ref[0]
        h = _gelu(h)
        cw = h * P2_ref[:, 1:2] + P2_ref[:, 2:3]
        mu = jnp.mean(cw, axis=0, keepdims=True)
        var = jnp.mean((cw - mu) ** 2, axis=0, keepdims=True)
        cwn = (cw - mu) / jnp.sqrt(var + 1e-5) * P2_ref[:, 3:4] + P2_ref[:, 4:5]
        fusionT = vqT * cwn
        h2 = lax.dot_general(PT1_ref[...], fusionT, (((1,), (0,)), ((), ())),
                             preferred_element_type=f32)
        h2 = _gelu(h2 + P3_ref[:, 0:1])
        sco = jnp.sum(h2 * P3_ref[:, 1:2], axis=0, keepdims=True) + sc_ref[1]
        sco = sco * 10.0
        cio = lax.broadcasted_iota(jnp.int32, (1, Np), 1)
        sco = jnp.where(cio < N, sco, -1e30)
        m0 = jnp.max(sco, axis=1, keepdims=True)
        ex = jnp.exp(sco - m0)
        pw_s[...] = ex / jnp.sum(ex, axis=1, keepdims=True)

    pwv = pw_s[:, pl.ds(j * CB, CB)]
    affv = aff_s[...]
    z = jnp.where(affv >= t, jnp.exp(affv - M) / S, 0.0)
    e = jnp.exp(z * pwv)
    cio2 = lax.broadcasted_iota(jnp.int32, (1, CB), 1) + j * CB
    e = jnp.where(cio2 < N, e, 0.0)
    part = lax.dot_general(e, vmt_ref[0], (((1,), (0,)), ((), ())),
                           preferred_element_type=f32)

    @pl.when(j == 0)
    def _():
        out_ref[0] = part

    @pl.when(j > 0)
    def _():
        out_ref[0] = out_ref[0] + part

    @pl.when(j == nblk - 1)
    def _():
        acc = out_ref[0]
        out_ref[0] = acc / acc[:, 16:17]


def _run_scale(kq, vq, mk, vm, p, H, W):
    f32 = jnp.float32
    N = H * W
    Np = -(-N // 128) * 128
    CB = min(512, Np)
    nblk = Np // CB

    kqf = kq.reshape(_B, _CK, N)
    vqf = vq.reshape(_B, _CV, N)

    qk = jnp.pad(kqf, ((0, 0), (0, 0), (0, Np - N)))
    mkt = jnp.pad(jnp.swapaxes(mk, 1, 2), ((0, 0), (0, Np - N), (0, 0)))
    vqT = jnp.pad(vqf, ((0, 0), (0, 0), (0, Np - N)))
    vmt = jnp.pad(jnp.swapaxes(vm, 1, 2), ((0, 0), (0, Np - N), (0, 0)))
    ones_col = jnp.ones((_B, Np, 1), f32)
    vmt_aug = jnp.concatenate(
        [vmt, ones_col, jnp.zeros((_B, Np, 15), f32)], axis=2)

    P2 = jnp.stack([p['mw1'][:, 0], p['mw2'][0, :], p['mb2'], p['g'], p['b'],
                    jnp.zeros((16,), f32), jnp.zeros((16,), f32),
                    jnp.zeros((16,), f32)], axis=1)
    PT1 = p['pw1'].T
    P3 = jnp.stack([p['pb1'], p['pw2'][:, 0]], axis=1)
    sc = jnp.stack([p['mb1'][0], p['pb2'][0]])

    body = functools.partial(_scale_body, N, Np, CB, nblk)
    out1 = pl.pallas_call(
        body,
        grid=(_B, nblk),
        in_specs=[
            pl.BlockSpec((1, Np, _CK), lambda b, j: (b, 0, 0)),
            pl.BlockSpec((1, _CK, CB), lambda b, j: (b, 0, j)),
            pl.BlockSpec((1, _CV, Np), lambda b, j: (b, 0, 0)),
            pl.BlockSpec((1, CB, 32), lambda b, j: (b, j, 0)),
            pl.BlockSpec((16, 8), lambda b, j: (0, 0)),
            pl.BlockSpec((8, 16), lambda b, j: (0, 0)),
            pl.BlockSpec((8, 2), lambda b, j: (0, 0)),
            pl.BlockSpec(memory_space=pltpu.SMEM),
        ],
        out_specs=pl.BlockSpec((1, Np, 32), lambda b, j: (b, 0, 0)),
        out_shape=jax.ShapeDtypeStruct((_B, Np, 32), f32),
        scratch_shapes=[
            pltpu.VMEM((Np, CB), f32),
            pltpu.VMEM((Np, CB), f32),
            pltpu.VMEM((1, Np), f32),
        ],
        compiler_params=pltpu.CompilerParams(
            dimension_semantics=("arbitrary", "arbitrary"),
        ),
    )(mkt, qk, vqT, vmt_aug, P2, PT1, P3, sc)

    ro = jnp.swapaxes(out1[:, :N, :16], 1, 2).reshape(_B, _CV, H, W)
    return jnp.concatenate([ro, vq], axis=1)


def kernel(keyQ0, keyQ1, keyQ2, valueQ0, valueQ1, valueQ2,
           keyM0, keyM1, keyM2, valueM0, valueM1, valueM2, params):
    keyQ = [keyQ0, keyQ1, keyQ2]
    valueQ = [valueQ0, valueQ1, valueQ2]
    keyM = [keyM0, keyM1, keyM2]
    valueM = [valueM0, valueM1, valueM2]
    outs = []
    for s, (H, W) in enumerate(_SCALES):
        outs.append(_run_scale(keyQ[s], valueQ[s], keyM[s], valueM[s],
                               params[f'p{s}'], H, W))
    return tuple(outs)


# speedup vs baseline: 18.2197x; 1.3797x over previous
import functools
import math

import jax
import jax.numpy as jnp
from jax import lax
from jax.experimental import pallas as pl
from jax.experimental.pallas import tpu as pltpu

_B = 8
_CK = 64
_CV = 16
_K = 20
_SCALES = [(44, 44), (22, 22), (11, 11)]

_SQRT2 = math.sqrt(2.0)


def _gelu(x):
    return x * 0.5 * (1.0 + lax.erf(x / _SQRT2))


def _scale_body(N, Np, CB, nblk,
                mkt_ref, qk_ref, vqT_ref, vmt_ref, P2_ref, PT1_ref, P3_ref,
                sc_ref, out_ref, aff_s, pw_s):
    j = pl.program_id(1)
    f32 = jnp.float32

    mkt = mkt_ref[0]
    lhs = jnp.concatenate([mkt * 0.25, (mkt * mkt) * (-0.125)], axis=1)
    qk = qk_ref[0]
    rhs = jnp.concatenate([qk, jnp.ones((_CK, CB), f32)], axis=0)
    aff = lax.dot_general(lhs, rhs, (((1,), (0,)), ((), ())),
                          preferred_element_type=f32)
    rio = lax.broadcasted_iota(jnp.int32, (Np, 1), 0)
    aff = jnp.where(rio < N, aff, -1e30)
    aff_s[...] = aff

    M = jnp.max(aff, axis=0, keepdims=True)

    def tk_body(_, c):
        S, t = c
        w = aff_s[...]
        cur = jnp.max(jnp.where(w < t, w, -jnp.inf), axis=0, keepdims=True)
        S = S + jnp.exp(cur - M)
        return (S, cur)

    S, t = lax.fori_loop(0, _K - 1, tk_body,
                         (jnp.ones((1, CB), f32), M))

    @pl.when(j == 0)
    def _():
        vqT = vqT_ref[0]
        pooled = jnp.sum(vqT, axis=1, keepdims=True) * (1.0 / N)
        h = jnp.sum(pooled * P2_ref[:, 0:1], axis=0, keepdims=True) + sc_ref[0]
        h = _gelu(h)
        cw = h * P2_ref[:, 1:2] + P2_ref[:, 2:3]
        mu = jnp.mean(cw, axis=0, keepdims=True)
        var = jnp.mean((cw - mu) ** 2, axis=0, keepdims=True)
        cwn = (cw - mu) / jnp.sqrt(var + 1e-5) * P2_ref[:, 3:4] + P2_ref[:, 4:5]
        fusionT = vqT * cwn
        h2 = lax.dot_general(PT1_ref[...], fusionT, (((1,), (0,)), ((), ())),
                             preferred_element_type=f32)
        h2 = _gelu(h2 + P3_ref[:, 0:1])
        sco = jnp.sum(h2 * P3_ref[:, 1:2], axis=0, keepdims=True) + sc_ref[1]
        sco = sco * 10.0
        cio = lax.broadcasted_iota(jnp.int32, (1, Np), 1)
        sco = jnp.where(cio < N, sco, -1e30)
        m0 = jnp.max(sco, axis=1, keepdims=True)
        ex = jnp.exp(sco - m0)
        pw_s[...] = ex / jnp.sum(ex, axis=1, keepdims=True)

    pwv = pw_s[:, pl.ds(j * CB, CB)]
    affv = aff_s[...]
    z = jnp.where(affv >= t, jnp.exp(affv - M) / S, 0.0)
    e = jnp.exp(z * pwv)
    cio2 = lax.broadcasted_iota(jnp.int32, (1, CB), 1) + j * CB
    e = jnp.where(cio2 < N, e, 0.0)
    part = lax.dot_general(e, vmt_ref[0], (((1,), (0,)), ((), ())),
                           preferred_element_type=f32)

    @pl.when(j == 0)
    def _():
        out_ref[0] = part

    @pl.when(j > 0)
    def _():
        out_ref[0] = out_ref[0] + part

    @pl.when(j == nblk - 1)
    def _():
        acc = out_ref[0]
        out_ref[0] = acc / acc[:, 16:17]


def _run_scale(kq, vq, mk, vm, p, H, W):
    f32 = jnp.float32
    N = H * W
    Np = -(-N // 128) * 128
    CB = min(512, Np)
    nblk = Np // CB

    kqf = kq.reshape(_B, _CK, N)
    vqf = vq.reshape(_B, _CV, N)

    qk = jnp.pad(kqf, ((0, 0), (0, 0), (0, Np - N)))
    mkt = jnp.pad(jnp.swapaxes(mk, 1, 2), ((0, 0), (0, Np - N), (0, 0)))
    vqT = jnp.pad(vqf, ((0, 0), (0, 0), (0, Np - N)))
    vmt = jnp.pad(jnp.swapaxes(vm, 1, 2), ((0, 0), (0, Np - N), (0, 0)))
    ones_col = jnp.ones((_B, Np, 1), f32)
    vmt_aug = jnp.concatenate(
        [vmt, ones_col, jnp.zeros((_B, Np, 15), f32)], axis=2)

    P2 = jnp.stack([p['mw1'][:, 0], p['mw2'][0, :], p['mb2'], p['g'], p['b'],
                    jnp.zeros((16,), f32), jnp.zeros((16,), f32),
                    jnp.zeros((16,), f32)], axis=1)
    PT1 = p['pw1'].T
    P3 = jnp.stack([p['pb1'], p['pw2'][:, 0]], axis=1)
    sc = jnp.stack([p['mb1'][0], p['pb2'][0]])

    body = functools.partial(_scale_body, N, Np, CB, nblk)
    out1 = pl.pallas_call(
        body,
        grid=(_B, nblk),
        in_specs=[
            pl.BlockSpec((1, Np, _CK), lambda b, j: (b, 0, 0)),
            pl.BlockSpec((1, _CK, CB), lambda b, j: (b, 0, j)),
            pl.BlockSpec((1, _CV, Np), lambda b, j: (b, 0, 0)),
            pl.BlockSpec((1, CB, 32), lambda b, j: (b, j, 0)),
            pl.BlockSpec((16, 8), lambda b, j: (0, 0)),
            pl.BlockSpec((8, 16), lambda b, j: (0, 0)),
            pl.BlockSpec((8, 2), lambda b, j: (0, 0)),
            pl.BlockSpec(memory_space=pltpu.SMEM),
        ],
        out_specs=pl.BlockSpec((1, Np, 32), lambda b, j: (b, 0, 0)),
        out_shape=jax.ShapeDtypeStruct((_B, Np, 32), f32),
        scratch_shapes=[
            pltpu.VMEM((Np, CB), f32),
            pltpu.VMEM((1, Np), f32),
        ],
        compiler_params=pltpu.CompilerParams(
            dimension_semantics=("arbitrary", "arbitrary"),
        ),
    )(mkt, qk, vqT, vmt_aug, P2, PT1, P3, sc)

    ro = jnp.swapaxes(out1[:, :N, :16], 1, 2).reshape(_B, _CV, H, W)
    return jnp.concatenate([ro, vq], axis=1)


def kernel(keyQ0, keyQ1, keyQ2, valueQ0, valueQ1, valueQ2,
           keyM0, keyM1, keyM2, valueM0, valueM1, valueM2, params):
    keyQ = [keyQ0, keyQ1, keyQ2]
    valueQ = [valueQ0, valueQ1, valueQ2]
    keyM = [keyM0, keyM1, keyM2]
    valueM = [valueM0, valueM1, valueM2]
    outs = []
    for s, (H, W) in enumerate(_SCALES):
        outs.append(_run_scale(keyQ[s], valueQ[s], keyM[s], valueM[s],
                               params[f'p{s}'], H, W))
    return tuple(outs)


# speedup vs baseline: 19.4881x; 1.0696x over previous
import functools
import math

import jax
import jax.numpy as jnp
from jax import lax
from jax.experimental import pallas as pl
from jax.experimental.pallas import tpu as pltpu

_B = 8
_CK = 64
_CV = 16
_K = 20
_SCALES = [(44, 44), (22, 22), (11, 11)]

_SQRT2 = math.sqrt(2.0)


def _gelu(x):
    return x * 0.5 * (1.0 + lax.erf(x / _SQRT2))


def _scale_body(N, Np, CB, nblk,
                mkt_ref, qk_ref, vqT_ref, vmt_ref, P2_ref, PT1_ref, P3_ref,
                sc_ref, out_ref, aff_s, pw_s, cap_s):
    j = pl.program_id(1)
    f32 = jnp.float32

    mkt = mkt_ref[0]
    lhs = jnp.concatenate([mkt * 0.25, (mkt * mkt) * (-0.125)], axis=1)
    qk = qk_ref[0]
    rhs = jnp.concatenate([qk, jnp.ones((_CK, CB), f32)], axis=0)
    aff = lax.dot_general(lhs, rhs, (((1,), (0,)), ((), ())),
                          preferred_element_type=f32)
    rio = lax.broadcasted_iota(jnp.int32, (Np, 1), 0)
    aff = jnp.where(rio < N, aff, -1e30)
    aff_s[...] = aff


    def _extract(ref):
        M0 = jnp.max(ref[...], axis=0, keepdims=True)

        def tk_body(_, c):
            S0, t0 = c
            w = ref[...]
            cur = jnp.max(jnp.where(w < t0, w, -jnp.inf), axis=0,
                          keepdims=True)
            return (S0 + jnp.exp(cur - M0), cur)

        S0, t0 = lax.fori_loop(0, _K - 1, tk_body,
                               (jnp.ones((1, CB), f32), M0))
        return M0, S0, t0

    if Np >= 1024:
        G = Np // 32
        m1 = jnp.max(aff.reshape(G, 32, CB), axis=1)
        cap_s[0:G, :] = m1

        def cap_body(r, cur):
            a3 = aff_s[...].reshape(G, 32, CB)
            nxt = jnp.max(jnp.where(a3 < cur[:, None, :], a3, -jnp.inf),
                          axis=1)
            cap_s[pl.ds((r + 1) * G, G), :] = nxt
            return nxt

        lax.fori_loop(0, 5, cap_body, m1)
        M, S, t = _extract(cap_s)
        cnt = jnp.sum(jnp.where(aff_s[...] >= t, 1.0, 0.0), axis=0,
                      keepdims=True)
        S, t = lax.cond(jnp.any(cnt != float(_K)),
                        lambda: _extract(aff_s)[1:],
                        lambda: (S, t))
    else:
        M, S, t = _extract(aff_s)

    @pl.when(j == 0)
    def _():
        vqT = vqT_ref[0]
        pooled = jnp.sum(vqT, axis=1, keepdims=True) * (1.0 / N)
        h = jnp.sum(pooled * P2_ref[:, 0:1], axis=0, keepdims=True) + sc_ref[0]
        h = _gelu(h)
        cw = h * P2_ref[:, 1:2] + P2_ref[:, 2:3]
        mu = jnp.mean(cw, axis=0, keepdims=True)
        var = jnp.mean((cw - mu) ** 2, axis=0, keepdims=True)
        cwn = (cw - mu) / jnp.sqrt(var + 1e-5) * P2_ref[:, 3:4] + P2_ref[:, 4:5]
        fusionT = vqT * cwn
        h2 = lax.dot_general(PT1_ref[...], fusionT, (((1,), (0,)), ((), ())),
                             preferred_element_type=f32)
        h2 = _gelu(h2 + P3_ref[:, 0:1])
        sco = jnp.sum(h2 * P3_ref[:, 1:2], axis=0, keepdims=True) + sc_ref[1]
        sco = sco * 10.0
        cio = lax.broadcasted_iota(jnp.int32, (1, Np), 1)
        sco = jnp.where(cio < N, sco, -1e30)
        m0 = jnp.max(sco, axis=1, keepdims=True)
        ex = jnp.exp(sco - m0)
        pw_s[...] = ex / jnp.sum(ex, axis=1, keepdims=True)

    pwv = pw_s[:, pl.ds(j * CB, CB)]
    affv = aff_s[...]
    z = jnp.where(affv >= t, jnp.exp(affv - M) / S, 0.0)
    e = jnp.exp(z * pwv)
    cio2 = lax.broadcasted_iota(jnp.int32, (1, CB), 1) + j * CB
    e = jnp.where(cio2 < N, e, 0.0)
    part = lax.dot_general(e, vmt_ref[0], (((1,), (0,)), ((), ())),
                           preferred_element_type=f32)

    @pl.when(j == 0)
    def _():
        out_ref[0] = part

    @pl.when(j > 0)
    def _():
        out_ref[0] = out_ref[0] + part

    @pl.when(j == nblk - 1)
    def _():
        acc = out_ref[0]
        out_ref[0] = acc / acc[:, 16:17]


def _run_scale(kq, vq, mk, vm, p, H, W):
    f32 = jnp.float32
    N = H * W
    Np = -(-N // 128) * 128
    CB = min(512, Np)
    nblk = Np // CB

    kqf = kq.reshape(_B, _CK, N)
    vqf = vq.reshape(_B, _CV, N)

    qk = jnp.pad(kqf, ((0, 0), (0, 0), (0, Np - N)))
    mkt = jnp.pad(jnp.swapaxes(mk, 1, 2), ((0, 0), (0, Np - N), (0, 0)))
    vqT = jnp.pad(vqf, ((0, 0), (0, 0), (0, Np - N)))
    vmt = jnp.pad(jnp.swapaxes(vm, 1, 2), ((0, 0), (0, Np - N), (0, 0)))
    ones_col = jnp.ones((_B, Np, 1), f32)
    vmt_aug = jnp.concatenate(
        [vmt, ones_col, jnp.zeros((_B, Np, 15), f32)], axis=2)

    P2 = jnp.stack([p['mw1'][:, 0], p['mw2'][0, :], p['mb2'], p['g'], p['b'],
                    jnp.zeros((16,), f32), jnp.zeros((16,), f32),
                    jnp.zeros((16,), f32)], axis=1)
    PT1 = p['pw1'].T
    P3 = jnp.stack([p['pb1'], p['pw2'][:, 0]], axis=1)
    sc = jnp.stack([p['mb1'][0], p['pb2'][0]])

    body = functools.partial(_scale_body, N, Np, CB, nblk)
    out1 = pl.pallas_call(
        body,
        grid=(_B, nblk),
        in_specs=[
            pl.BlockSpec((1, Np, _CK), lambda b, j: (b, 0, 0)),
            pl.BlockSpec((1, _CK, CB), lambda b, j: (b, 0, j)),
            pl.BlockSpec((1, _CV, Np), lambda b, j: (b, 0, 0)),
            pl.BlockSpec((1, CB, 32), lambda b, j: (b, j, 0)),
            pl.BlockSpec((16, 8), lambda b, j: (0, 0)),
            pl.BlockSpec((8, 16), lambda b, j: (0, 0)),
            pl.BlockSpec((8, 2), lambda b, j: (0, 0)),
            pl.BlockSpec(memory_space=pltpu.SMEM),
        ],
        out_specs=pl.BlockSpec((1, Np, 32), lambda b, j: (b, 0, 0)),
        out_shape=jax.ShapeDtypeStruct((_B, Np, 32), f32),
        scratch_shapes=[
            pltpu.VMEM((Np, CB), f32),
            pltpu.VMEM((1, Np), f32),
            pltpu.VMEM((max(6 * (Np // 32), 8), CB), f32),
        ],
        compiler_params=pltpu.CompilerParams(
            dimension_semantics=("arbitrary", "arbitrary"),
        ),
    )(mkt, qk, vqT, vmt_aug, P2, PT1, P3, sc)

    ro = jnp.swapaxes(out1[:, :N, :16], 1, 2).reshape(_B, _CV, H, W)
    return jnp.concatenate([ro, vq], axis=1)


def kernel(keyQ0, keyQ1, keyQ2, valueQ0, valueQ1, valueQ2,
           keyM0, keyM1, keyM2, valueM0, valueM1, valueM2, params):
    keyQ = [keyQ0, keyQ1, keyQ2]
    valueQ = [valueQ0, valueQ1, valueQ2]
    keyM = [keyM0, keyM1, keyM2]
    valueM = [valueM0, valueM1, valueM2]
    outs = []
    for s, (H, W) in enumerate(_SCALES):
        outs.append(_run_scale(keyQ[s], valueQ[s], keyM[s], valueM[s],
                               params[f'p{s}'], H, W))
    return tuple(outs)


# speedup vs baseline: 22.1589x; 1.1371x over previous
import functools
import math

import jax
import jax.numpy as jnp
from jax import lax
from jax.experimental import pallas as pl
from jax.experimental.pallas import tpu as pltpu

_B = 8
_CK = 64
_CV = 16
_K = 20
_SCALES = [(44, 44), (22, 22), (11, 11)]

_SQRT2 = math.sqrt(2.0)


def _gelu(x):
    return x * 0.5 * (1.0 + lax.erf(x / _SQRT2))


def _scale_body(N, Np, CB, nblk,
                mkt_ref, qk_ref, vqT_ref, vmt_ref, P2_ref, PT1_ref, P3_ref,
                sc_ref, out_ref, aff_s, pw_s, cap_s):
    j = pl.program_id(1)
    f32 = jnp.float32

    mkt = mkt_ref[0]
    lhs = jnp.concatenate([mkt * 0.25, (mkt * mkt) * (-0.125)], axis=1)
    qk = qk_ref[0]
    rhs = jnp.concatenate([qk, jnp.ones((_CK, CB), f32)], axis=0)
    aff = lax.dot_general(lhs, rhs, (((1,), (0,)), ((), ())),
                          preferred_element_type=f32)
    rio = lax.broadcasted_iota(jnp.int32, (Np, 1), 0)
    aff = jnp.where(rio < N, aff, -1e30)
    aff_s[...] = aff


    def _extract(ref):
        M0 = jnp.max(ref[...], axis=0, keepdims=True)

        def tk_body(_, c):
            S0, t0 = c
            w = ref[...]
            cur = jnp.max(jnp.where(w < t0, w, -jnp.inf), axis=0,
                          keepdims=True)
            return (S0 + jnp.exp(cur - M0), cur)

        S0, t0 = lax.fori_loop(0, _K - 1, tk_body,
                               (jnp.ones((1, CB), f32), M0))
        return M0, S0, t0

    def _fold(x, op):
        while x.shape[0] > 64:
            h = x.shape[0] // 2
            x = op(x[:h], x[h:])
        return x

    if Np >= 1024:
        reps = Np // 64
        m1 = _fold(aff, jnp.maximum)
        cap_s[0:64, :] = m1

        def cap_body(r, cur):
            a = aff_s[...]
            curb = jnp.broadcast_to(cur, (reps, 64, CB)).reshape(Np, CB)
            nxt = _fold(jnp.where(a < curb, a, -jnp.inf), jnp.maximum)
            cap_s[pl.ds((r + 1) * 64, 64), :] = nxt
            return nxt

        lax.fori_loop(0, 5, cap_body, m1)
        M, S, t = _extract(cap_s)
        ind = jnp.where(aff_s[...] >= t, 1.0, 0.0)
        cnt = jnp.sum(_fold(ind, jnp.add), axis=0, keepdims=True)
        S, t = lax.cond(jnp.any(cnt != float(_K)),
                        lambda: _extract(aff_s)[1:],
                        lambda: (S, t))
    else:
        M, S, t = _extract(aff_s)

    @pl.when(j == 0)
    def _():
        vqT = vqT_ref[0]
        pooled = jnp.sum(vqT, axis=1, keepdims=True) * (1.0 / N)
        h = jnp.sum(pooled * P2_ref[:, 0:1], axis=0, keepdims=True) + sc_ref[0]
        h = _gelu(h)
        cw = h * P2_ref[:, 1:2] + P2_ref[:, 2:3]
        mu = jnp.mean(cw, axis=0, keepdims=True)
        var = jnp.mean((cw - mu) ** 2, axis=0, keepdims=True)
        cwn = (cw - mu) / jnp.sqrt(var + 1e-5) * P2_ref[:, 3:4] + P2_ref[:, 4:5]
        fusionT = vqT * cwn
        h2 = lax.dot_general(PT1_ref[...], fusionT, (((1,), (0,)), ((), ())),
                             preferred_element_type=f32)
        h2 = _gelu(h2 + P3_ref[:, 0:1])
        sco = jnp.sum(h2 * P3_ref[:, 1:2], axis=0, keepdims=True) + sc_ref[1]
        sco = sco * 10.0
        cio = lax.broadcasted_iota(jnp.int32, (1, Np), 1)
        sco = jnp.where(cio < N, sco, -1e30)
        m0 = jnp.max(sco, axis=1, keepdims=True)
        ex = jnp.exp(sco - m0)
        pw_s[...] = ex / jnp.sum(ex, axis=1, keepdims=True)

    pwv = pw_s[:, pl.ds(j * CB, CB)]
    affv = aff_s[...]
    z = jnp.where(affv >= t, jnp.exp(affv - M) / S, 0.0)
    e = jnp.exp(z * pwv)
    cio2 = lax.broadcasted_iota(jnp.int32, (1, CB), 1) + j * CB
    e = jnp.where(cio2 < N, e, 0.0)
    part = lax.dot_general(e, vmt_ref[0], (((1,), (0,)), ((), ())),
                           preferred_element_type=f32)

    @pl.when(j == 0)
    def _():
        out_ref[0] = part

    @pl.when(j > 0)
    def _():
        out_ref[0] = out_ref[0] + part

    @pl.when(j == nblk - 1)
    def _():
        acc = out_ref[0]
        out_ref[0] = acc / acc[:, 16:17]


def _run_scale(kq, vq, mk, vm, p, H, W):
    f32 = jnp.float32
    N = H * W
    Np = -(-N // 128) * 128
    CB = min(512, Np)
    nblk = Np // CB

    kqf = kq.reshape(_B, _CK, N)
    vqf = vq.reshape(_B, _CV, N)

    qk = jnp.pad(kqf, ((0, 0), (0, 0), (0, Np - N)))
    mkt = jnp.pad(jnp.swapaxes(mk, 1, 2), ((0, 0), (0, Np - N), (0, 0)))
    vqT = jnp.pad(vqf, ((0, 0), (0, 0), (0, Np - N)))
    vmt = jnp.pad(jnp.swapaxes(vm, 1, 2), ((0, 0), (0, Np - N), (0, 0)))
    ones_col = jnp.ones((_B, Np, 1), f32)
    vmt_aug = jnp.concatenate(
        [vmt, ones_col, jnp.zeros((_B, Np, 15), f32)], axis=2)

    P2 = jnp.stack([p['mw1'][:, 0], p['mw2'][0, :], p['mb2'], p['g'], p['b'],
                    jnp.zeros((16,), f32), jnp.zeros((16,), f32),
                    jnp.zeros((16,), f32)], axis=1)
    PT1 = p['pw1'].T
    P3 = jnp.stack([p['pb1'], p['pw2'][:, 0]], axis=1)
    sc = jnp.stack([p['mb1'][0], p['pb2'][0]])

    body = functools.partial(_scale_body, N, Np, CB, nblk)
    out1 = pl.pallas_call(
        body,
        grid=(_B, nblk),
        in_specs=[
            pl.BlockSpec((1, Np, _CK), lambda b, j: (b, 0, 0)),
            pl.BlockSpec((1, _CK, CB), lambda b, j: (b, 0, j)),
            pl.BlockSpec((1, _CV, Np), lambda b, j: (b, 0, 0)),
            pl.BlockSpec((1, CB, 32), lambda b, j: (b, j, 0)),
            pl.BlockSpec((16, 8), lambda b, j: (0, 0)),
            pl.BlockSpec((8, 16), lambda b, j: (0, 0)),
            pl.BlockSpec((8, 2), lambda b, j: (0, 0)),
            pl.BlockSpec(memory_space=pltpu.SMEM),
        ],
        out_specs=pl.BlockSpec((1, Np, 32), lambda b, j: (b, 0, 0)),
        out_shape=jax.ShapeDtypeStruct((_B, Np, 32), f32),
        scratch_shapes=[
            pltpu.VMEM((Np, CB), f32),
            pltpu.VMEM((1, Np), f32),
            pltpu.VMEM((384 if Np >= 1024 else 8, CB), f32),
        ],
        compiler_params=pltpu.CompilerParams(
            dimension_semantics=("arbitrary", "arbitrary"),
        ),
    )(mkt, qk, vqT, vmt_aug, P2, PT1, P3, sc)

    ro = jnp.swapaxes(out1[:, :N, :16], 1, 2).reshape(_B, _CV, H, W)
    return jnp.concatenate([ro, vq], axis=1)


def kernel(keyQ0, keyQ1, keyQ2, valueQ0, valueQ1, valueQ2,
           keyM0, keyM1, keyM2, valueM0, valueM1, valueM2, params):
    keyQ = [keyQ0, keyQ1, keyQ2]
    valueQ = [valueQ0, valueQ1, valueQ2]
    keyM = [keyM0, keyM1, keyM2]
    valueM = [valueM0, valueM1, valueM2]
    outs = []
    for s, (H, W) in enumerate(_SCALES):
        outs.append(_run_scale(keyQ[s], valueQ[s], keyM[s], valueM[s],
                               params[f'p{s}'], H, W))
    return tuple(outs)
